# baseline (speedup 1.0000x reference)
import functools

import jax
import jax.numpy as jnp
from jax import lax
from jax.experimental import pallas as pl
from jax.experimental.pallas import tpu as pltpu
from jax.experimental.pallas import tpu_sc as plsc

N_FIELDS = 26
EMB_DIM = 19
BATCH = 4096
USER_DIM = 128
ALL_DIM = USER_DIM + N_FIELDS * EMB_DIM
VOCAB = 100000

NC = 2
NS = 16
L = 16
NW = NC * NS
BPW = BATCH // NW
NG = BPW // L

PROJ_CHUNK = 8192
N_CHUNKS = (VOCAB + PROJ_CHUNK - 1) // PROJ_CHUNK
SEG = N_FIELDS * PROJ_CHUNK
V_FLAT = N_CHUNKS * SEG


def _proj_body(w_ref, *refs):
    out_ref = refs[-1]
    for i in range(N_FIELDS):
        t = refs[i][...]
        w = w_ref[i, :]
        out_ref[pl.ds(i * PROJ_CHUNK, PROJ_CHUNK)] = jnp.sum(
            t * w[:, None], axis=0)


_proj = pl.pallas_call(
    _proj_body,
    grid=(N_CHUNKS,),
    in_specs=[pl.BlockSpec((N_FIELDS, EMB_DIM), lambda j: (0, 0))]
    + [pl.BlockSpec((EMB_DIM, PROJ_CHUNK), lambda j: (0, j))
       for _ in range(N_FIELDS)],
    out_specs=pl.BlockSpec((SEG,), lambda j: (j,)),
    out_shape=jax.ShapeDtypeStruct((V_FLAT,), jnp.float32),
)


def _sc_body(user_hbm, feats_hbm, wu_hbm, b_hbm, v_hbm, out_hbm,
             u_v, fa_v, vg, w_v, b_v, out_v, sem):
    wid = lax.axis_index("s") * NC + lax.axis_index("c")
    base = wid * BPW

    pltpu.sync_copy(wu_hbm, w_v)
    pltpu.sync_copy(b_hbm, b_v)
    pltpu.sync_copy(feats_hbm.at[:, pl.ds(base, BPW)], fa_v)
    pltpu.sync_copy(user_hbm.at[pl.ds(base, BPW), :], u_v)

    cps = [
        pltpu.async_copy(v_hbm.at[fa_v.at[i]], vg.at[i], sem)
        for i in range(N_FIELDS)
    ]

    lanes = lax.iota(jnp.int32, 16)
    n16 = [jnp.full((L,), g * L, jnp.int32) + lanes for g in range(NG)]
    accs = tuple(jnp.zeros((L,), jnp.float32) for _ in range(NG))

    def ubody(d, accs):
        accs = list(accs)
        dvec = jnp.full((L,), d, jnp.int32)
        wv = plsc.load_gather(w_v, [dvec])
        for g in range(NG):
            val = plsc.load_gather(u_v, [n16[g], dvec])
            accs[g] = accs[g] + val * wv
        return tuple(accs)

    accs = lax.fori_loop(0, USER_DIM, ubody, accs)

    for cp in cps:
        cp.wait()

    bvec = b_v[...]
    for g in range(NG):
        s = accs[g]
        for i in range(N_FIELDS):
            s = s + vg[i, pl.ds(g * L, L)]
        x = s + bvec
        out_v[pl.ds(g * L, L)] = 1.0 / (1.0 + jnp.exp(-x))
    pltpu.sync_copy(out_v, out_hbm.at[pl.ds(base, BPW)])


def _sc_scratch():
    return [
        pltpu.VMEM((BPW, USER_DIM), jnp.float32),
        pltpu.VMEM((N_FIELDS, BPW), jnp.int32),
        pltpu.VMEM((N_FIELDS, BPW), jnp.float32),
        pltpu.VMEM((USER_DIM,), jnp.float32),
        pltpu.VMEM((L,), jnp.float32),
        pltpu.VMEM((BPW,), jnp.float32),
        pltpu.SemaphoreType.DMA,
    ]


@jax.jit
def _run(user_feature, feats_adj, wu, wE, b16, *tables):
    tTs = [t.T for t in tables]
    v1 = _proj(wE, *tTs)

    mesh = plsc.VectorSubcoreMesh(core_axis_name="c", subcore_axis_name="s")
    fn = pl.kernel(
        _sc_body,
        out_type=jax.ShapeDtypeStruct((BATCH,), jnp.float32),
        mesh=mesh,
        scratch_types=_sc_scratch(),
        compiler_params=pltpu.CompilerParams(
            needs_layout_passes=False, use_tc_tiling_on_sc=False),
    )
    return fn(user_feature, feats_adj, wu, b16, v1)


def kernel(user_feature, feat_0, feat_1, feat_2, feat_3, feat_4, feat_5, feat_6, feat_7, feat_8, feat_9, feat_10, feat_11, feat_12, feat_13, feat_14, feat_15, feat_16, feat_17, feat_18, feat_19, feat_20, feat_21, feat_22, feat_23, feat_24, feat_25, table_0, table_1, table_2, table_3, table_4, table_5, table_6, table_7, table_8, table_9, table_10, table_11, table_12, table_13, table_14, table_15, table_16, table_17, table_18, table_19, table_20, table_21, table_22, table_23, table_24, table_25, W, b):
    feats_list = [feat_0, feat_1, feat_2, feat_3, feat_4, feat_5, feat_6, feat_7, feat_8, feat_9, feat_10, feat_11, feat_12, feat_13, feat_14, feat_15, feat_16, feat_17, feat_18, feat_19, feat_20, feat_21, feat_22, feat_23, feat_24, feat_25]
    tables = [table_0, table_1, table_2, table_3, table_4, table_5, table_6, table_7, table_8, table_9, table_10, table_11, table_12, table_13, table_14, table_15, table_16, table_17, table_18, table_19, table_20, table_21, table_22, table_23, table_24, table_25]
    offs = (jnp.arange(N_FIELDS, dtype=jnp.int32) * PROJ_CHUNK)[:, None]
    fa = jnp.stack(feats_list, axis=0)
    feats_adj = (fa // PROJ_CHUNK) * SEG + offs + (fa % PROJ_CHUNK)
    wu = W[:USER_DIM, 0]
    wE = W[USER_DIM:ALL_DIM, 0].reshape(N_FIELDS, EMB_DIM)
    b16 = jnp.broadcast_to(b, (L,))
    return _run(user_feature, feats_adj, wu, wE, b16, *tables)


# speedup vs baseline: 1.0353x; 1.0353x over previous
import functools

import jax
import jax.numpy as jnp
from jax import lax
from jax.experimental import pallas as pl
from jax.experimental.pallas import tpu as pltpu
from jax.experimental.pallas import tpu_sc as plsc

N_FIELDS = 26
EMB_DIM = 19
BATCH = 4096
USER_DIM = 128
ALL_DIM = USER_DIM + N_FIELDS * EMB_DIM
VOCAB = 100000

NC = 2
NS = 16
L = 16
NW = NC * NS
BPW = BATCH // NW
NG = BPW // L

PROJ_CHUNK = 4096
N_CHUNKS = (VOCAB + PROJ_CHUNK - 1) // PROJ_CHUNK
SEG = N_FIELDS * PROJ_CHUNK
V_FLAT = N_CHUNKS * SEG


def _proj_body(w_ref, *refs):
    out_ref = refs[-1]
    for i in range(N_FIELDS):
        t = refs[i][...]
        w = w_ref[i, :]
        out_ref[pl.ds(i * PROJ_CHUNK, PROJ_CHUNK)] = jnp.sum(
            t * w[:, None], axis=0)


_proj = pl.pallas_call(
    _proj_body,
    grid=(N_CHUNKS,),
    in_specs=[pl.BlockSpec((N_FIELDS, EMB_DIM), lambda j: (0, 0))]
    + [pl.BlockSpec((EMB_DIM, PROJ_CHUNK), lambda j: (0, j))
       for _ in range(N_FIELDS)],
    out_specs=pl.BlockSpec((SEG,), lambda j: (j,)),
    out_shape=jax.ShapeDtypeStruct((V_FLAT,), jnp.float32),
)


def _sc_body(user_hbm, feats_hbm, wu_hbm, b_hbm, v_hbm, out_hbm,
             u_v, fa_v, vg, w_v, b_v, out_v, sem):
    wid = lax.axis_index("s") * NC + lax.axis_index("c")
    base = wid * BPW

    pltpu.sync_copy(wu_hbm, w_v)
    pltpu.sync_copy(b_hbm, b_v)
    pltpu.sync_copy(feats_hbm.at[:, pl.ds(base, BPW)], fa_v)
    pltpu.sync_copy(user_hbm.at[pl.ds(base, BPW), :], u_v)

    cps = [
        pltpu.async_copy(v_hbm.at[fa_v.at[i]], vg.at[i], sem)
        for i in range(N_FIELDS)
    ]

    lanes = lax.iota(jnp.int32, 16)
    n16 = [jnp.full((L,), g * L, jnp.int32) + lanes for g in range(NG)]
    accs = tuple(jnp.zeros((L,), jnp.float32) for _ in range(NG))

    def ubody(d, accs):
        accs = list(accs)
        dvec = jnp.full((L,), d, jnp.int32)
        wv = plsc.load_gather(w_v, [dvec])
        for g in range(NG):
            val = plsc.load_gather(u_v, [n16[g], dvec])
            accs[g] = accs[g] + val * wv
        return tuple(accs)

    accs = lax.fori_loop(0, USER_DIM, ubody, accs)

    for cp in cps:
        cp.wait()

    bvec = b_v[...]
    for g in range(NG):
        s = accs[g]
        for i in range(N_FIELDS):
            s = s + vg[i, pl.ds(g * L, L)]
        x = s + bvec
        out_v[pl.ds(g * L, L)] = 1.0 / (1.0 + jnp.exp(-x))
    pltpu.sync_copy(out_v, out_hbm.at[pl.ds(base, BPW)])


def _sc_scratch():
    return [
        pltpu.VMEM((BPW, USER_DIM), jnp.float32),
        pltpu.VMEM((N_FIELDS, BPW), jnp.int32),
        pltpu.VMEM((N_FIELDS, BPW), jnp.float32),
        pltpu.VMEM((USER_DIM,), jnp.float32),
        pltpu.VMEM((L,), jnp.float32),
        pltpu.VMEM((BPW,), jnp.float32),
        pltpu.SemaphoreType.DMA,
    ]


@jax.jit
def _run(user_feature, feats_adj, wu, wE, b16, *tables):
    tTs = [t.T for t in tables]
    v1 = _proj(wE, *tTs)

    mesh = plsc.VectorSubcoreMesh(core_axis_name="c", subcore_axis_name="s")
    fn = pl.kernel(
        _sc_body,
        out_type=jax.ShapeDtypeStruct((BATCH,), jnp.float32),
        mesh=mesh,
        scratch_types=_sc_scratch(),
        compiler_params=pltpu.CompilerParams(
            needs_layout_passes=False, use_tc_tiling_on_sc=False),
    )
    return fn(user_feature, feats_adj, wu, b16, v1)


def kernel(user_feature, feat_0, feat_1, feat_2, feat_3, feat_4, feat_5, feat_6, feat_7, feat_8, feat_9, feat_10, feat_11, feat_12, feat_13, feat_14, feat_15, feat_16, feat_17, feat_18, feat_19, feat_20, feat_21, feat_22, feat_23, feat_24, feat_25, table_0, table_1, table_2, table_3, table_4, table_5, table_6, table_7, table_8, table_9, table_10, table_11, table_12, table_13, table_14, table_15, table_16, table_17, table_18, table_19, table_20, table_21, table_22, table_23, table_24, table_25, W, b):
    feats_list = [feat_0, feat_1, feat_2, feat_3, feat_4, feat_5, feat_6, feat_7, feat_8, feat_9, feat_10, feat_11, feat_12, feat_13, feat_14, feat_15, feat_16, feat_17, feat_18, feat_19, feat_20, feat_21, feat_22, feat_23, feat_24, feat_25]
    tables = [table_0, table_1, table_2, table_3, table_4, table_5, table_6, table_7, table_8, table_9, table_10, table_11, table_12, table_13, table_14, table_15, table_16, table_17, table_18, table_19, table_20, table_21, table_22, table_23, table_24, table_25]
    offs = (jnp.arange(N_FIELDS, dtype=jnp.int32) * PROJ_CHUNK)[:, None]
    fa = jnp.stack(feats_list, axis=0)
    feats_adj = (fa // PROJ_CHUNK) * SEG + offs + (fa % PROJ_CHUNK)
    wu = W[:USER_DIM, 0]
    wE = W[USER_DIM:ALL_DIM, 0].reshape(N_FIELDS, EMB_DIM)
    b16 = jnp.broadcast_to(b, (L,))
    return _run(user_feature, feats_adj, wu, wE, b16, *tables)
